# baseline (speedup 1.0000x reference)
import functools

import jax
import jax.numpy as jnp
from jax import lax
from jax.experimental import pallas as pl
from jax.experimental.pallas import tpu as pltpu
from jax.experimental.pallas import tpu_sc as plsc

N_NODES = 10000
N_EDGES = 320000
IN_CH = 128
HID_CH = 256
OUT_CH = 128

NC = 2
NS = 16
W = 80
EPT = N_EDGES // NS
NW = EPT // W
EPT_DEG = N_EDGES // (NC * NS)
NW_DEG = EPT_DEG // W
N_PAD = 10240
RPT = N_PAD // NS
WB = 128

@functools.lru_cache(maxsize=None)
def _vec_mesh():
    return plsc.VectorSubcoreMesh(core_axis_name="c", subcore_axis_name="s")


DEG_PAD = NC * NS * 640


def _sc_degree(dst4):
    @functools.partial(
        pl.kernel,
        out_type=jax.ShapeDtypeStruct((NC, DEG_PAD // NC), jnp.float32),
        mesh=_vec_mesh(),
        scratch_types=[
            pltpu.VMEM((NW_DEG, W), jnp.int32),
            pltpu.VMEM((W,), jnp.float32),
            pltpu.VMEM((640,), jnp.float32),
            pltpu.VMEM_SHARED((DEG_PAD // NC,), jnp.float32),
        ],
    )
    def deg_kernel(dst_hbm, deg_out, idx_v, ones_v, zbuf, deg_sh):
        c = lax.axis_index("c")
        s = lax.axis_index("s")

        @pl.loop(0, 640, step=16)
        def _(i):
            zbuf[pl.ds(i, 16)] = jnp.zeros((16,), jnp.float32)

        for k in range(W // 16):
            ones_v[pl.ds(k * 16, 16)] = jnp.ones((16,), jnp.float32)

        pltpu.sync_copy(zbuf, deg_sh.at[pl.ds(s * 640, 640)])
        pltpu.sync_copy(dst_hbm.at[c, s], idx_v)
        plsc.subcore_barrier()

        @pl.loop(0, NW_DEG)
        def _(w):
            pltpu.sync_copy(ones_v, deg_sh.at[idx_v.at[w]], add=True)

        plsc.subcore_barrier()
        pltpu.sync_copy(deg_sh.at[pl.ds(s * 640, 640)], zbuf)
        pltpu.sync_copy(zbuf, deg_out.at[c, pl.ds(s * 640, 640)])

    return deg_kernel(dst4)


CS = 128
CH = 8
NW1 = 256
NW2 = 128


@functools.lru_cache(maxsize=None)
def _make_sc_aggregate(nw):
    cs = CS
    nch = nw // CH

    @functools.partial(
        pl.kernel,
        out_type=jax.ShapeDtypeStruct((NC, N_PAD, cs), jnp.float32),
        mesh=_vec_mesh(),
        scratch_types=[
            pltpu.VMEM((2, CH, W), jnp.int32),
            pltpu.VMEM((2, CH, W), jnp.int32),
            pltpu.VMEM((2, W, cs), jnp.float32),
            pltpu.VMEM_SHARED((N_PAD, cs), jnp.float32),
            pltpu.SemaphoreType.DMA,
            pltpu.SemaphoreType.DMA,
            pltpu.SemaphoreType.DMA,
            pltpu.SemaphoreType.DMA,
        ],
    )
    def agg_kernel(g_hbm, src_hbm, dst_hbm, z_hbm, acc_hbm,
                   src_i, dst_i, rows_v, acc_sh, gsem0, gsem1, isem0, isem1):
        c = lax.axis_index("c")
        s = lax.axis_index("s")
        gsems = (gsem0, gsem1)
        isems = (isem0, isem1)

        pltpu.sync_copy(z_hbm.at[pl.ds(0, W)], rows_v.at[0])
        for k in range(RPT // W):
            r0 = s * RPT + k * W
            pltpu.sync_copy(rows_v.at[0], acc_sh.at[pl.ds(r0, W)])
        plsc.subcore_barrier()

        def load_idx(ch, ib):
            pltpu.async_copy(src_hbm.at[c, s, ch], src_i.at[ib], isems[ib])
            pltpu.async_copy(dst_hbm.at[c, s, ch], dst_i.at[ib], isems[ib])

        def wait_idx(ch, ib):
            pltpu.make_async_copy(src_hbm.at[c, s, ch], src_i.at[ib],
                                  isems[ib]).wait()
            pltpu.make_async_copy(dst_hbm.at[c, s, ch], dst_i.at[ib],
                                  isems[ib]).wait()

        H = W // 2

        def start_gather(ib, j, b):
            pltpu.async_copy(g_hbm.at[src_i.at[ib, j, pl.ds(0, H)]],
                             rows_v.at[b, pl.ds(0, H)], gsems[b])
            pltpu.async_copy(g_hbm.at[src_i.at[ib, j, pl.ds(H, H)]],
                             rows_v.at[b, pl.ds(H, H)], gsems[b])

        def wait_gather(ib, j, b):
            pltpu.make_async_copy(g_hbm.at[src_i.at[ib, j, pl.ds(0, H)]],
                                  rows_v.at[b, pl.ds(0, H)], gsems[b]).wait()
            pltpu.make_async_copy(g_hbm.at[src_i.at[ib, j, pl.ds(H, H)]],
                                  rows_v.at[b, pl.ds(H, H)], gsems[b]).wait()

        load_idx(0, 0)
        wait_idx(0, 0)
        load_idx(1, 1)
        start_gather(0, 0, 0)
        start_gather(0, 1, 1)

        @pl.loop(0, nch, step=2)
        def _(ch2):
          for ib in range(2):
            ch = ch2 + ib
            for j in range(CH):
                b = j % 2
                wait_gather(ib, j, b)
                pltpu.sync_copy(rows_v.at[b], acc_sh.at[dst_i.at[ib, j]],
                                add=True)
                if j < CH - 2:
                    start_gather(ib, j + 2, b)
                elif j == CH - 2:
                    @pl.when(ch + 1 < nch)
                    def _():
                        wait_idx(ch + 1, 1 - ib)
                        start_gather(1 - ib, 0, b)
                else:
                    @pl.when(ch + 1 < nch)
                    def _():
                        start_gather(1 - ib, 1, b)

                    @pl.when(ch + 2 < nch)
                    def _():
                        load_idx(ch + 2, ib)

        plsc.subcore_barrier()
        for k in range(RPT // W):
            r0 = s * RPT + k * W
            pltpu.sync_copy(acc_sh.at[pl.ds(r0, W)], rows_v.at[0])
            pltpu.sync_copy(rows_v.at[0], acc_hbm.at[c, pl.ds(r0, W)])

    return agg_kernel


def _pad_tiles(a, per, pad_vals):
    lead = a.shape[:-1]
    nt = a.shape[-1] // per
    a = a.reshape(lead + (nt, per))
    pad = jnp.broadcast_to(pad_vals, lead + (nt, pad_vals.shape[0]))
    return jnp.concatenate([a, pad], axis=-1)


_RB = 1000


def _tc_layer1(x, w1, degp):
    def body(x_ref, w1_ref, degp_ref, o_ref):
        deg = degp_ref[:, 0] + degp_ref[:, 1] + 1.0
        dinv = lax.rsqrt(deg)[:, None]
        h = jnp.dot(x_ref[...], w1_ref[...], preferred_element_type=jnp.float32,
                    precision=lax.Precision.HIGHEST)
        g = h * dinv
        o_ref[0] = g[:, : HID_CH // 2]
        o_ref[1] = g[:, HID_CH // 2:]

    return pl.pallas_call(
        body,
        grid=(N_NODES // _RB,),
        in_specs=[
            pl.BlockSpec((_RB, IN_CH), lambda i: (i, 0)),
            pl.BlockSpec((IN_CH, HID_CH), lambda i: (0, 0)),
            pl.BlockSpec((_RB, NC), lambda i: (i, 0)),
        ],
        out_specs=pl.BlockSpec((NC, _RB, HID_CH // 2), lambda i: (0, i, 0)),
        out_shape=jax.ShapeDtypeStruct((NC, N_NODES, HID_CH // 2), jnp.float32),
    )(x, w1, degp)


def _tc_layer2(acc1, g1, degp, w2, b1):
    def body(acc_ref, g_ref, degp_ref, w2_ref, b1_ref, o_ref):
        deg = degp_ref[:, 0] + degp_ref[:, 1] + 1.0
        dinv = lax.rsqrt(deg)[:, None]
        t0 = (acc_ref[0] + g_ref[0]) * dinv
        t1 = (acc_ref[1] + g_ref[1]) * dinv
        out1 = jnp.concatenate([t0, t1], axis=1) + b1_ref[...]
        out1 = jnp.maximum(out1, 0.0)
        h2 = jnp.dot(out1, w2_ref[...], preferred_element_type=jnp.float32,
                     precision=lax.Precision.HIGHEST)
        o_ref[...] = h2 * dinv

    return pl.pallas_call(
        body,
        grid=(N_NODES // _RB,),
        in_specs=[
            pl.BlockSpec((NC, _RB, HID_CH // 2), lambda i: (0, i, 0)),
            pl.BlockSpec((NC, _RB, HID_CH // 2), lambda i: (0, i, 0)),
            pl.BlockSpec((_RB, NC), lambda i: (i, 0)),
            pl.BlockSpec((HID_CH, OUT_CH), lambda i: (0, 0)),
            pl.BlockSpec((1, HID_CH), lambda i: (0, 0)),
        ],
        out_specs=pl.BlockSpec((_RB, OUT_CH), lambda i: (i, 0)),
        out_shape=jax.ShapeDtypeStruct((N_NODES, OUT_CH), jnp.float32),
    )(acc1, g1, degp, w2, b1)


def _tc_final(acc2, g2, degp, b2):
    def body(acc_ref, g_ref, degp_ref, b2_ref, o_ref):
        deg = degp_ref[:, 0] + degp_ref[:, 1] + 1.0
        dinv = lax.rsqrt(deg)[:, None]
        o_ref[...] = (acc_ref[0] + acc_ref[1] + g_ref[...]) * dinv + b2_ref[...]

    return pl.pallas_call(
        body,
        grid=(N_NODES // _RB,),
        in_specs=[
            pl.BlockSpec((NC, _RB, OUT_CH), lambda i: (0, i, 0)),
            pl.BlockSpec((_RB, OUT_CH), lambda i: (i, 0)),
            pl.BlockSpec((_RB, NC), lambda i: (i, 0)),
            pl.BlockSpec((1, OUT_CH), lambda i: (0, 0)),
        ],
        out_specs=pl.BlockSpec((_RB, OUT_CH), lambda i: (i, 0)),
        out_shape=jax.ShapeDtypeStruct((N_NODES, OUT_CH), jnp.float32),
    )(acc2, g2, degp, b2)


def kernel(x, edge_index, W1, b1, W2, b2):
    src = edge_index[0]
    dst = edge_index[1]
    zeros2 = jnp.zeros((W, CS), jnp.float32)

    dst_deg = dst.reshape(NC, NS, NW_DEG, W)

    pad1 = NW1 * W - N_EDGES // NS
    pad_src = (jnp.arange(pad1, dtype=jnp.int32) * 37) % N_NODES
    pad_dst = N_NODES + (jnp.arange(pad1, dtype=jnp.int32) % (N_PAD - N_NODES))
    src_t = _pad_tiles(src, N_EDGES // NS, pad_src)
    dst_t = _pad_tiles(dst, N_EDGES // NS, pad_dst)
    src_l1 = jnp.stack([src_t, src_t + N_NODES]).reshape(NC, NS, NW1 // CH, CH, W)
    dst_l1 = jnp.stack([dst_t, dst_t]).reshape(NC, NS, NW1 // CH, CH, W)
    pad2 = NW2 * W - N_EDGES // (NC * NS)
    src_l2 = _pad_tiles(src.reshape(NC, -1), N_EDGES // (NC * NS),
                        pad_src[:pad2]).reshape(NC, NS, NW2 // CH, CH, W)
    dst_l2 = _pad_tiles(dst.reshape(NC, -1), N_EDGES // (NC * NS),
                        pad_dst[:pad2]).reshape(NC, NS, NW2 // CH, CH, W)

    degp_pad = _sc_degree(dst_deg)
    degp = degp_pad[:, :N_NODES].T

    g1 = _tc_layer1(x, W1, degp)
    acc1 = _make_sc_aggregate(NW1)(
        g1.reshape(NC * N_NODES, CS), src_l1, dst_l1, zeros2)[:, :N_NODES]

    g2 = _tc_layer2(acc1, g1, degp, W2, b1.reshape(1, HID_CH))
    acc2 = _make_sc_aggregate(NW2)(
        g2, src_l2, dst_l2, zeros2)[:, :N_NODES]

    return _tc_final(acc2, g2, degp, b2.reshape(1, OUT_CH))


# speedup vs baseline: 1.0051x; 1.0051x over previous
import functools

import jax
import jax.numpy as jnp
from jax import lax
from jax.experimental import pallas as pl
from jax.experimental.pallas import tpu as pltpu
from jax.experimental.pallas import tpu_sc as plsc

N_NODES = 10000
N_EDGES = 320000
IN_CH = 128
HID_CH = 256
OUT_CH = 128

NC = 2
NS = 16
W = 80
EPT = N_EDGES // NS
NW = EPT // W
EPT_DEG = N_EDGES // (NC * NS)
NW_DEG = EPT_DEG // W
N_PAD = 10240
RPT = N_PAD // NS
WB = 128

@functools.lru_cache(maxsize=None)
def _vec_mesh():
    return plsc.VectorSubcoreMesh(core_axis_name="c", subcore_axis_name="s")


DEG_PAD = NC * NS * 640


def _sc_degree(dst4):
    @functools.partial(
        pl.kernel,
        out_type=jax.ShapeDtypeStruct((NC, DEG_PAD // NC), jnp.float32),
        mesh=_vec_mesh(),
        scratch_types=[
            pltpu.VMEM((NW_DEG, W), jnp.int32),
            pltpu.VMEM((W,), jnp.float32),
            pltpu.VMEM((640,), jnp.float32),
            pltpu.VMEM_SHARED((DEG_PAD // NC,), jnp.float32),
        ],
    )
    def deg_kernel(dst_hbm, deg_out, idx_v, ones_v, zbuf, deg_sh):
        c = lax.axis_index("c")
        s = lax.axis_index("s")

        @pl.loop(0, 640, step=16)
        def _(i):
            zbuf[pl.ds(i, 16)] = jnp.zeros((16,), jnp.float32)

        for k in range(W // 16):
            ones_v[pl.ds(k * 16, 16)] = jnp.ones((16,), jnp.float32)

        pltpu.sync_copy(zbuf, deg_sh.at[pl.ds(s * 640, 640)])
        pltpu.sync_copy(dst_hbm.at[c, s], idx_v)
        plsc.subcore_barrier()

        @pl.loop(0, NW_DEG)
        def _(w):
            pltpu.sync_copy(ones_v, deg_sh.at[idx_v.at[w]], add=True)

        plsc.subcore_barrier()
        pltpu.sync_copy(deg_sh.at[pl.ds(s * 640, 640)], zbuf)
        pltpu.sync_copy(zbuf, deg_out.at[c, pl.ds(s * 640, 640)])

    return deg_kernel(dst4)


CS = 128
CH = 8
NW1 = 256
NW2 = 128


@functools.lru_cache(maxsize=None)
def _make_sc_aggregate(nw):
    cs = CS
    nch = nw // CH

    @functools.partial(
        pl.kernel,
        out_type=jax.ShapeDtypeStruct((NC, N_PAD, cs), jnp.float32),
        mesh=_vec_mesh(),
        scratch_types=[
            pltpu.VMEM((2, CH, W), jnp.int32),
            pltpu.VMEM((2, CH, W), jnp.int32),
            pltpu.VMEM((2, W, cs), jnp.float32),
            pltpu.VMEM_SHARED((N_PAD, cs), jnp.float32),
            pltpu.SemaphoreType.DMA,
            pltpu.SemaphoreType.DMA,
            pltpu.SemaphoreType.DMA,
            pltpu.SemaphoreType.DMA,
        ],
    )
    def agg_kernel(g_hbm, src_hbm, dst_hbm, z_hbm, acc_hbm,
                   src_i, dst_i, rows_v, acc_sh, gsem0, gsem1, isem0, isem1):
        c = lax.axis_index("c")
        s = lax.axis_index("s")
        gsems = (gsem0, gsem1)
        isems = (isem0, isem1)

        pltpu.sync_copy(z_hbm.at[pl.ds(0, W)], rows_v.at[0])
        for k in range(RPT // W):
            r0 = s * RPT + k * W
            pltpu.sync_copy(rows_v.at[0], acc_sh.at[pl.ds(r0, W)])
        plsc.subcore_barrier()

        def load_idx(ch, ib):
            pltpu.async_copy(src_hbm.at[c, s, ch], src_i.at[ib], isems[ib])
            pltpu.async_copy(dst_hbm.at[c, s, ch], dst_i.at[ib], isems[ib])

        def wait_idx(ch, ib):
            pltpu.make_async_copy(src_hbm.at[c, s, ch], src_i.at[ib],
                                  isems[ib]).wait()
            pltpu.make_async_copy(dst_hbm.at[c, s, ch], dst_i.at[ib],
                                  isems[ib]).wait()

        def start_gather(ib, j, b):
            pltpu.async_copy(g_hbm.at[src_i.at[ib, j]], rows_v.at[b], gsems[b])

        def wait_gather(ib, j, b):
            pltpu.make_async_copy(g_hbm.at[src_i.at[ib, j]], rows_v.at[b],
                                  gsems[b]).wait()

        load_idx(0, 0)
        wait_idx(0, 0)
        load_idx(1, 1)
        start_gather(0, 0, 0)
        start_gather(0, 1, 1)

        @pl.loop(0, nch, step=2)
        def _(ch2):
          for ib in range(2):
            ch = ch2 + ib
            for j in range(CH):
                b = j % 2
                wait_gather(ib, j, b)
                pltpu.sync_copy(rows_v.at[b], acc_sh.at[dst_i.at[ib, j]],
                                add=True)
                if j < CH - 2:
                    start_gather(ib, j + 2, b)
                elif j == CH - 2:
                    @pl.when(ch + 1 < nch)
                    def _():
                        wait_idx(ch + 1, 1 - ib)
                        start_gather(1 - ib, 0, b)
                else:
                    @pl.when(ch + 1 < nch)
                    def _():
                        start_gather(1 - ib, 1, b)

                    @pl.when(ch + 2 < nch)
                    def _():
                        load_idx(ch + 2, ib)

        plsc.subcore_barrier()
        for k in range(RPT // W):
            r0 = s * RPT + k * W
            pltpu.sync_copy(acc_sh.at[pl.ds(r0, W)], rows_v.at[0])
            pltpu.sync_copy(rows_v.at[0], acc_hbm.at[c, pl.ds(r0, W)])

    return agg_kernel


def _pad_tiles(a, per, pad_vals):
    lead = a.shape[:-1]
    nt = a.shape[-1] // per
    a = a.reshape(lead + (nt, per))
    pad = jnp.broadcast_to(pad_vals, lead + (nt, pad_vals.shape[0]))
    return jnp.concatenate([a, pad], axis=-1)


_RB = 1000


def _tc_layer1(x, w1, degp):
    def body(x_ref, w1_ref, degp_ref, o_ref):
        deg = degp_ref[:, 0] + degp_ref[:, 1] + 1.0
        dinv = lax.rsqrt(deg)[:, None]
        h = jnp.dot(x_ref[...], w1_ref[...], preferred_element_type=jnp.float32,
                    precision=lax.Precision.HIGHEST)
        g = h * dinv
        o_ref[0] = g[:, : HID_CH // 2]
        o_ref[1] = g[:, HID_CH // 2:]

    return pl.pallas_call(
        body,
        grid=(N_NODES // _RB,),
        in_specs=[
            pl.BlockSpec((_RB, IN_CH), lambda i: (i, 0)),
            pl.BlockSpec((IN_CH, HID_CH), lambda i: (0, 0)),
            pl.BlockSpec((_RB, NC), lambda i: (i, 0)),
        ],
        out_specs=pl.BlockSpec((NC, _RB, HID_CH // 2), lambda i: (0, i, 0)),
        out_shape=jax.ShapeDtypeStruct((NC, N_NODES, HID_CH // 2), jnp.float32),
    )(x, w1, degp)


def _tc_layer2(acc1, g1, degp, w2, b1):
    def body(acc_ref, g_ref, degp_ref, w2_ref, b1_ref, o_ref):
        deg = degp_ref[:, 0] + degp_ref[:, 1] + 1.0
        dinv = lax.rsqrt(deg)[:, None]
        t0 = (acc_ref[0] + g_ref[0]) * dinv
        t1 = (acc_ref[1] + g_ref[1]) * dinv
        out1 = jnp.concatenate([t0, t1], axis=1) + b1_ref[...]
        out1 = jnp.maximum(out1, 0.0)
        h2 = jnp.dot(out1, w2_ref[...], preferred_element_type=jnp.float32,
                     precision=lax.Precision.HIGHEST)
        o_ref[...] = h2 * dinv

    return pl.pallas_call(
        body,
        grid=(N_NODES // _RB,),
        in_specs=[
            pl.BlockSpec((NC, _RB, HID_CH // 2), lambda i: (0, i, 0)),
            pl.BlockSpec((NC, _RB, HID_CH // 2), lambda i: (0, i, 0)),
            pl.BlockSpec((_RB, NC), lambda i: (i, 0)),
            pl.BlockSpec((HID_CH, OUT_CH), lambda i: (0, 0)),
            pl.BlockSpec((1, HID_CH), lambda i: (0, 0)),
        ],
        out_specs=pl.BlockSpec((_RB, OUT_CH), lambda i: (i, 0)),
        out_shape=jax.ShapeDtypeStruct((N_NODES, OUT_CH), jnp.float32),
    )(acc1, g1, degp, w2, b1)


def _tc_final(acc2, g2, degp, b2):
    def body(acc_ref, g_ref, degp_ref, b2_ref, o_ref):
        deg = degp_ref[:, 0] + degp_ref[:, 1] + 1.0
        dinv = lax.rsqrt(deg)[:, None]
        o_ref[...] = (acc_ref[0] + acc_ref[1] + g_ref[...]) * dinv + b2_ref[...]

    return pl.pallas_call(
        body,
        grid=(N_NODES // _RB,),
        in_specs=[
            pl.BlockSpec((NC, _RB, OUT_CH), lambda i: (0, i, 0)),
            pl.BlockSpec((_RB, OUT_CH), lambda i: (i, 0)),
            pl.BlockSpec((_RB, NC), lambda i: (i, 0)),
            pl.BlockSpec((1, OUT_CH), lambda i: (0, 0)),
        ],
        out_specs=pl.BlockSpec((_RB, OUT_CH), lambda i: (i, 0)),
        out_shape=jax.ShapeDtypeStruct((N_NODES, OUT_CH), jnp.float32),
    )(acc2, g2, degp, b2)


def kernel(x, edge_index, W1, b1, W2, b2):
    src = edge_index[0]
    dst = edge_index[1]
    zeros2 = jnp.zeros((W, CS), jnp.float32)

    dst_deg = dst.reshape(NC, NS, NW_DEG, W)

    pad1 = NW1 * W - N_EDGES // NS
    pad_src = (jnp.arange(pad1, dtype=jnp.int32) * 37) % N_NODES
    pad_dst = N_NODES + (jnp.arange(pad1, dtype=jnp.int32) % (N_PAD - N_NODES))
    src_t = _pad_tiles(src, N_EDGES // NS, pad_src)
    dst_t = _pad_tiles(dst, N_EDGES // NS, pad_dst)
    src_l1 = jnp.stack([src_t, src_t + N_NODES]).reshape(NC, NS, NW1 // CH, CH, W)
    dst_l1 = jnp.stack([dst_t, dst_t]).reshape(NC, NS, NW1 // CH, CH, W)
    pad2 = NW2 * W - N_EDGES // (NC * NS)
    src_l2 = _pad_tiles(src.reshape(NC, -1), N_EDGES // (NC * NS),
                        pad_src[:pad2]).reshape(NC, NS, NW2 // CH, CH, W)
    dst_l2 = _pad_tiles(dst.reshape(NC, -1), N_EDGES // (NC * NS),
                        pad_dst[:pad2]).reshape(NC, NS, NW2 // CH, CH, W)

    degp_pad = _sc_degree(dst_deg)
    degp = degp_pad[:, :N_NODES].T

    g1 = _tc_layer1(x, W1, degp)
    acc1 = _make_sc_aggregate(NW1)(
        g1.reshape(NC * N_NODES, CS), src_l1, dst_l1, zeros2)[:, :N_NODES]

    g2 = _tc_layer2(acc1, g1, degp, W2, b1.reshape(1, HID_CH))
    acc2 = _make_sc_aggregate(NW2)(
        g2, src_l2, dst_l2, zeros2)[:, :N_NODES]

    return _tc_final(acc2, g2, degp, b2.reshape(1, OUT_CH))
